# Initial kernel scaffold; baseline (speedup 1.0000x reference)
import jax
import jax.numpy as jnp
from jax.experimental import pallas as pl


def kernel(x, y):
    raise NotImplementedError("write your pallas kernel here")


# speedup vs baseline: 4.1791x; 4.1791x over previous
import jax
import jax.numpy as jnp
from jax.experimental import pallas as pl

_K = 64
_BT = 512


def _body(x_ref, y_ref, o_ref):
    lab = x_ref[...]
    keep = (lab >= 0) & (lab < _K)
    o_ref[...] = jnp.where(keep, y_ref[...], 0.0)


def kernel(x, y):
    n, d = y.shape
    xi = x.astype(jnp.int32).reshape(n, 1)
    return pl.pallas_call(
        _body,
        grid=(n // _BT,),
        in_specs=[
            pl.BlockSpec((_BT, 1), lambda i: (i, 0)),
            pl.BlockSpec((_BT, d), lambda i: (i, 0)),
        ],
        out_specs=pl.BlockSpec((_BT, d), lambda i: (i, 0)),
        out_shape=jax.ShapeDtypeStruct((n, d), jnp.float32),
    )(xi, y)
